# baseline (speedup 1.0000x reference)
import functools

import jax
import jax.numpy as jnp
from jax import lax
from jax.experimental import pallas as pl
from jax.experimental.pallas import tpu as pltpu
from jax.experimental.pallas import tpu_sc as plsc

_NUM_BS = 256
_NUM_UE = 8
_EMB_DIM = 64
_BATCH = 16384
_TAB_ROWS = _NUM_BS * _NUM_UE // 2
_TAB_W = 2 * _EMB_DIM

_INFO = plsc.get_sparse_core_info()
_NC = _INFO.num_cores
_NS = _INFO.num_subcores
_L = _INFO.num_lanes
_NW = _NC * _NS
_BPW = _BATCH // _NW
_CHUNK = 128
_NCHUNK = _BPW // _CHUNK

_mesh = plsc.VectorSubcoreMesh(core_axis_name="c", subcore_axis_name="s")


@functools.partial(
    pl.kernel,
    out_type=jax.ShapeDtypeStruct((_BATCH, _EMB_DIM), jnp.float32),
    mesh=_mesh,
    scratch_types=[
        pltpu.VMEM((_BPW,), jnp.int32),
        pltpu.VMEM((_BPW,), jnp.int32),
        pltpu.VMEM((_BPW,), jnp.int32),
        pltpu.VMEM((_NCHUNK, _CHUNK), jnp.int32),
        pltpu.VMEM((2 * _CHUNK, _TAB_W), jnp.float32),
        pltpu.VMEM((_BPW, _EMB_DIM), jnp.float32),
        pltpu.SemaphoreType.DMA,
        pltpu.SemaphoreType.DMA((_NCHUNK,)),
        pltpu.SemaphoreType.DMA,
    ],
    compiler_params=pltpu.CompilerParams(use_tc_tiling_on_sc=True,
                                         needs_layout_passes=False),
)
def _gather_kernel(bs_hbm, ue_hbm, tab_hbm, out_hbm,
                   bs_v, ue_v, pair_v, idxrow_v, rows2_v, out_v,
                   sem_in, sem_g, sem_o):
    wid = lax.axis_index("s") * _NC + lax.axis_index("c")
    base = wid * _BPW
    iota = lax.iota(jnp.int32, _L)
    cp_b = pltpu.async_copy(bs_hbm.at[pl.ds(base, _BPW)], bs_v, sem_in)
    cp_u = pltpu.async_copy(ue_hbm.at[pl.ds(base, _BPW)], ue_v, sem_in)
    cp_b.wait()
    cp_u.wait()
    def fire(j):
        return pltpu.async_copy(
            tab_hbm.at[idxrow_v.at[j]],
            rows2_v.at[pl.ds((j % 2) * _CHUNK, _CHUNK)],
            sem_g.at[j])

    gathers = {}
    for j in range(_NCHUNK):
        for c in range(_CHUNK // _L):
            i = j * (_CHUNK // _L) + c
            b = bs_v[pl.ds(i * _L, _L)]
            u = ue_v[pl.ds(i * _L, _L)]
            p = b * _NUM_UE + u
            pair_v[pl.ds(i * _L, _L)] = p
            idxrow_v[j, pl.ds(c * _L, _L)] = lax.shift_right_logical(p, 1)
        if j < 2:
            gathers[j] = fire(j)
    outs = []
    for j in range(_NCHUNK):
        gathers[j].wait()
        slot = (j % 2) * _CHUNK
        for blk in range(_CHUNK // _L):
            i0 = j * _CHUNK + blk * _L
            rowv = i0 + iota
            srow = slot + blk * _L + iota
            h64 = (pair_v[pl.ds(i0, _L)] & 1) * _EMB_DIM

            @plsc.parallel_loop(0, _EMB_DIM, 1, unroll=8)
            def _half_select(c, rowv=rowv, srow=srow, h64=h64):
                vals = plsc.load_gather(rows2_v, [srow, h64 + c])
                plsc.store_scatter(out_v, [rowv, jnp.full((_L,), c, jnp.int32)],
                                   vals)
        if j + 2 < _NCHUNK:
            gathers[j + 2] = fire(j + 2)
        outs.append(
            pltpu.async_copy(out_v.at[pl.ds(j * _CHUNK, _CHUNK)],
                             out_hbm.at[pl.ds(base + j * _CHUNK, _CHUNK)],
                             sem_o))
    for cp in outs:
        cp.wait()


def kernel(bs_antenna_indices, ue_antenna_indices, embeddings):
    pair_table = embeddings.reshape(_TAB_ROWS, _TAB_W)
    return _gather_kernel(bs_antenna_indices.astype(jnp.int32),
                          ue_antenna_indices.astype(jnp.int32),
                          pair_table)


# speedup vs baseline: 1.2328x; 1.2328x over previous
import functools

import jax
import jax.numpy as jnp
from jax import lax
from jax.experimental import pallas as pl
from jax.experimental.pallas import tpu as pltpu
from jax.experimental.pallas import tpu_sc as plsc

_NUM_BS = 256
_NUM_UE = 8
_EMB_DIM = 64
_BATCH = 16384
_TAB_ROWS = _NUM_BS * _NUM_UE // 2
_TAB_W = 2 * _EMB_DIM

_INFO = plsc.get_sparse_core_info()
_NC = _INFO.num_cores
_NS = _INFO.num_subcores
_L = _INFO.num_lanes
_NW = _NC * _NS
_BPW = _BATCH // _NW
_CHUNK = 128
_NCHUNK = _BPW // _CHUNK

_mesh = plsc.VectorSubcoreMesh(core_axis_name="c", subcore_axis_name="s")


@functools.partial(
    pl.kernel,
    out_type=jax.ShapeDtypeStruct((_EMB_DIM, _BATCH), jnp.float32),
    mesh=_mesh,
    scratch_types=[
        pltpu.VMEM((_BPW,), jnp.int32),
        pltpu.VMEM((_BPW,), jnp.int32),
        pltpu.VMEM((_BPW,), jnp.int32),
        pltpu.VMEM((_NCHUNK, _CHUNK), jnp.int32),
        pltpu.VMEM((2 * _CHUNK, _TAB_W), jnp.float32),
        pltpu.VMEM((_EMB_DIM, _BPW), jnp.float32),
        pltpu.SemaphoreType.DMA,
        pltpu.SemaphoreType.DMA((_NCHUNK,)),
        pltpu.SemaphoreType.DMA,
    ],
    compiler_params=pltpu.CompilerParams(use_tc_tiling_on_sc=True,
                                         needs_layout_passes=False),
)
def _gather_kernel(bs_hbm, ue_hbm, tab_hbm, out_hbm,
                   bs_v, ue_v, pair_v, idxrow_v, rows2_v, outt_v,
                   sem_in, sem_g, sem_o):
    wid = lax.axis_index("s") * _NC + lax.axis_index("c")
    base = wid * _BPW
    iota = lax.iota(jnp.int32, _L)
    cp_b = pltpu.async_copy(bs_hbm.at[pl.ds(base, _BPW)], bs_v, sem_in)
    cp_u = pltpu.async_copy(ue_hbm.at[pl.ds(base, _BPW)], ue_v, sem_in)
    cp_b.wait()
    cp_u.wait()

    def fire(j):
        return pltpu.async_copy(
            tab_hbm.at[idxrow_v.at[j]],
            rows2_v.at[pl.ds((j % 2) * _CHUNK, _CHUNK)],
            sem_g.at[j])

    gathers = {}
    for j in range(_NCHUNK):
        for c in range(_CHUNK // _L):
            i = j * (_CHUNK // _L) + c
            b = bs_v[pl.ds(i * _L, _L)]
            u = ue_v[pl.ds(i * _L, _L)]
            p = b * _NUM_UE + u
            pair_v[pl.ds(i * _L, _L)] = p
            idxrow_v[j, pl.ds(c * _L, _L)] = lax.shift_right_logical(p, 1)
        if j < 2:
            gathers[j] = fire(j)
    outs = []
    for j in range(_NCHUNK):
        gathers[j].wait()
        slot = (j % 2) * _CHUNK
        for blk in range(_CHUNK // _L):
            i0 = j * _CHUNK + blk * _L
            srow = slot + blk * _L + iota
            h64 = (pair_v[pl.ds(i0, _L)] & 1) * _EMB_DIM

            @plsc.parallel_loop(0, _EMB_DIM, 1, unroll=8)
            def _half_select(c, i0=i0, srow=srow, h64=h64):
                vals = plsc.load_gather(rows2_v, [srow, h64 + c])
                outt_v[c, pl.ds(i0, _L)] = vals
        if j + 2 < _NCHUNK:
            gathers[j + 2] = fire(j + 2)
        outs.append(
            pltpu.async_copy(
                outt_v.at[:, pl.ds(j * _CHUNK, _CHUNK)],
                out_hbm.at[:, pl.ds(base + j * _CHUNK, _CHUNK)],
                sem_o))
    for cp in outs:
        cp.wait()


def kernel(bs_antenna_indices, ue_antenna_indices, embeddings):
    pair_table = embeddings.reshape(_TAB_ROWS, _TAB_W)
    out_t = _gather_kernel(bs_antenna_indices.astype(jnp.int32),
                           ue_antenna_indices.astype(jnp.int32),
                           pair_table)
    return out_t.T
